# Initial kernel scaffold; baseline (speedup 1.0000x reference)
import jax
import jax.numpy as jnp
from jax.experimental import pallas as pl


def kernel(hidden, pos, mask, W, b):
    raise NotImplementedError("write your pallas kernel here")


# speedup vs baseline: 130.3857x; 130.3857x over previous
import functools

import jax
import jax.numpy as jnp
from jax.experimental import pallas as pl

GRID = 8
NB = 32.0
NCELLS = GRID * GRID


def _tc_body(pxr, pyr, maskr, pxc, pyc, maskc, hid, wr, b2, out_ref, *, bi, a):
    i0 = pl.program_id(0) * bi
    px_i = pxc[pl.ds(i0, bi), :]
    py_i = pyc[pl.ds(i0, bi), :]
    m_i = maskc[pl.ds(i0, bi), :]

    relx = pxr[...] - px_i
    rely = pyr[...] - py_i

    col = jnp.clip(jnp.floor((relx + NB) * (1.0 / (2.0 * NB / GRID))).astype(jnp.int32), 0, GRID - 1)
    row = jnp.clip(jnp.floor((rely + NB) * (1.0 / (2.0 * NB / GRID))).astype(jnp.int32), 0, GRID - 1)
    within = (jnp.abs(relx) < NB) & (jnp.abs(rely) < NB)
    jr = jax.lax.broadcasted_iota(jnp.int32, (bi, a), 1)
    ir = jax.lax.broadcasted_iota(jnp.int32, (bi, a), 0) + i0
    valid = within & (jr != ir) & (maskr[...] > 0.0)
    cells = jnp.where(valid, row * GRID + col, NCELLS)

    hid_b = hid[...]

    def step(c, acc):
        ohc = (cells == c).astype(jnp.bfloat16)
        grid_c = jnp.dot(ohc, hid_b, preferred_element_type=jnp.float32)
        wc = wr[pl.ds(c, 1), :, :][0]
        return acc + jnp.dot(grid_c.astype(jnp.bfloat16), wc,
                             preferred_element_type=jnp.float32)

    acc = jnp.zeros(out_ref.shape, jnp.float32)
    acc = jax.lax.fori_loop(0, NCELLS, step, acc)
    out_ref[...] = (acc + b2[...]) * m_i


def kernel(hidden, pos, mask, W, b):
    a, h = hidden.shape
    bi = 256 if a % 256 == 0 else a
    mask_f = mask.astype(jnp.float32)
    pxr = pos[:, 0].reshape(1, a)
    pyr = pos[:, 1].reshape(1, a)
    maskr = mask_f.reshape(1, a)
    pxc = pos[:, 0].reshape(a, 1)
    pyc = pos[:, 1].reshape(a, 1)
    maskc = mask_f.reshape(a, 1)
    hid_b = hidden.astype(jnp.bfloat16)
    wr = W.reshape(h, NCELLS, h).transpose(1, 2, 0).astype(jnp.bfloat16)
    b2 = b.reshape(1, h)

    full = lambda s: pl.BlockSpec(s, lambda i: tuple(0 for _ in s))
    return pl.pallas_call(
        functools.partial(_tc_body, bi=bi, a=a),
        grid=(a // bi,),
        in_specs=[
            full((1, a)), full((1, a)), full((1, a)),
            full((a, 1)), full((a, 1)), full((a, 1)),
            full((a, h)),
            full((NCELLS, h, h)),
            full((1, h)),
        ],
        out_specs=pl.BlockSpec((bi, h), lambda i: (i, 0)),
        out_shape=jax.ShapeDtypeStruct((a, h), jnp.float32),
    )(pxr, pyr, maskr, pxc, pyc, maskc, hid_b, wr, b2)


# speedup vs baseline: 256.5468x; 1.9676x over previous
import functools

import jax
import jax.numpy as jnp
from jax.experimental import pallas as pl
from jax.experimental.pallas import tpu as pltpu

GRID = 8
NB = 32.0
NCELLS = GRID * GRID
INV_CELL = 1.0 / (2.0 * NB / GRID)


def _tc_body(pxr, pyr, maskr, pxc, pyc, maskc, hid, wt, b2, out_ref, scratch,
             *, bi, a, h):
    i0 = pl.program_id(0) * bi
    px_i = pxc[pl.ds(i0, bi), :]
    py_i = pyc[pl.ds(i0, bi), :]
    m_i = maskc[pl.ds(i0, bi), :]

    relx = pxr[...] - px_i
    rely = pyr[...] - py_i

    colf = jnp.clip(jnp.floor((relx + NB) * INV_CELL), 0.0, GRID - 1.0)
    rowf = jnp.clip(jnp.floor((rely + NB) * INV_CELL), 0.0, GRID - 1.0)
    within = (jnp.abs(relx) < NB) & (jnp.abs(rely) < NB)
    jr = jax.lax.broadcasted_iota(jnp.int32, (bi, a), 1)
    ir = jax.lax.broadcasted_iota(jnp.int32, (bi, a), 0) + i0
    valid = within & (jr != ir)
    vm = jnp.where(valid, 1.0, 0.0) * maskr[...]

    row_oh = [(jnp.where(rowf == float(r), 1.0, 0.0) * vm).astype(jnp.bfloat16)
              for r in range(GRID)]
    col_oh = [jnp.where(colf == float(c), 1.0, 0.0).astype(jnp.bfloat16)
              for c in range(GRID)]

    hid_b = hid[...]
    for cell in range(NCELLS):
        r, c = cell // GRID, cell % GRID
        oh = row_oh[r] * col_oh[c]
        grid_c = jnp.dot(oh, hid_b, preferred_element_type=jnp.float32)
        scratch[:, cell * h:(cell + 1) * h] = grid_c.astype(jnp.bfloat16)

    acc = jnp.dot(scratch[...], wt[...], preferred_element_type=jnp.float32)
    out_ref[...] = (acc + b2[...]) * m_i


def kernel(hidden, pos, mask, W, b):
    a, h = hidden.shape
    bi = 256 if a % 256 == 0 else a
    mask_f = mask.astype(jnp.float32)
    pxr = pos[:, 0].reshape(1, a)
    pyr = pos[:, 1].reshape(1, a)
    maskr = mask_f.reshape(1, a)
    pxc = pos[:, 0].reshape(a, 1)
    pyc = pos[:, 1].reshape(a, 1)
    maskc = mask_f.reshape(a, 1)
    hid_b = hidden.astype(jnp.bfloat16)
    wt = W.T.astype(jnp.bfloat16)
    b2 = b.reshape(1, h)

    full = lambda s: pl.BlockSpec(s, lambda i: tuple(0 for _ in s))
    return pl.pallas_call(
        functools.partial(_tc_body, bi=bi, a=a, h=h),
        grid=(a // bi,),
        in_specs=[
            full((1, a)), full((1, a)), full((1, a)),
            full((a, 1)), full((a, 1)), full((a, 1)),
            full((a, h)),
            full((NCELLS * h, h)),
            full((1, h)),
        ],
        out_specs=pl.BlockSpec((bi, h), lambda i: (i, 0)),
        out_shape=jax.ShapeDtypeStruct((a, h), jnp.float32),
        scratch_shapes=[pltpu.VMEM((bi, NCELLS * h), jnp.bfloat16)],
    )(pxr, pyr, maskr, pxc, pyc, maskc, hid_b, wt, b2)
